# Initial kernel scaffold; baseline (speedup 1.0000x reference)
import jax
import jax.numpy as jnp
from jax.experimental import pallas as pl


def kernel(user_node_id, movie_node_id, movie_genres, edge_index_um, edge_index_mu, user_emb, movie_emb, proj_W, proj_b, conv1_um_Wl, conv1_um_Wr, conv1_um_b, conv1_mu_Wl, conv1_mu_Wr, conv1_mu_b, conv2_um_Wl, conv2_um_Wr, conv2_um_b, conv2_mu_Wl, conv2_mu_Wr, conv2_mu_b):
    raise NotImplementedError("write your pallas kernel here")


# speedup vs baseline: 2.4908x; 2.4908x over previous
import functools

import jax
import jax.numpy as jnp
from jax import lax
from jax.experimental import pallas as pl
from jax.experimental.pallas import tpu as pltpu
from jax.experimental.pallas import tpu_sc as plsc

N = 25000
D = 128
NG = 16
E = 500000
HALF = 64
NS = 16
NPAD = 25088
ROWS_PT = NPAD // NS
FCH = 112
B = 80
NBATCH = 392
EPT = B * NBATCH
E_PAD = EPT * NS

_SC_PARAMS = pltpu.CompilerParams(use_tc_tiling_on_sc=False)


def _mesh():
    return plsc.VectorSubcoreMesh(core_axis_name="c", subcore_axis_name="s",
                                  num_cores=2, num_subcores=NS)


def _make_sc_agg():
    out_type = (jax.ShapeDtypeStruct((2, NPAD, HALF), jnp.float32),) * 2

    scratch_types = (
        pltpu.VMEM((B,), jnp.int32),
        pltpu.VMEM((B,), jnp.int32),
        pltpu.VMEM((B,), jnp.int32),
        pltpu.VMEM((B, HALF), jnp.float32),
        pltpu.VMEM((FCH, HALF), jnp.float32),
        pltpu.SemaphoreType.DMA,
        pltpu.VMEM_SHARED((NPAD, HALF), jnp.float32),
    )

    def body(xA, srcA, dstA, xB, srcB, dstB, aggA, aggB,
             idx_s, idx_d, idx2, rows, fb, sem, acc):
        c = lax.axis_index("c")
        s = lax.axis_index("s")
        my_row0 = s * ROWS_PT

        zeros16 = jnp.zeros((16,), jnp.float32)

        def phase(x_hbm, src_hbm, dst_hbm, agg_out):
            def initrow(r, _):
                for j in range(HALF // 16):
                    fb[r, pl.ds(16 * j, 16)] = zeros16
                return 0
            lax.fori_loop(0, FCH, initrow, 0)

            def zchunk(j, _):
                pltpu.sync_copy(fb, acc.at[pl.ds(my_row0 + j * FCH, FCH), :])
                return 0
            lax.fori_loop(0, ROWS_PT // FCH, zchunk, 0)
            plsc.subcore_barrier()

            base = s * EPT

            def bstep(i, _):
                e0 = base + i * B
                pltpu.sync_copy(src_hbm.at[pl.ds(e0, B)], idx_s)
                pltpu.sync_copy(dst_hbm.at[pl.ds(e0, B)], idx_d)
                for k in range(B // 16):
                    sl = pl.ds(16 * k, 16)
                    idx2[sl] = idx_s[sl] * 2 + c
                pltpu.async_copy(x_hbm.at[idx2], rows, sem).wait()
                pltpu.sync_copy(rows, acc.at[idx_d], add=True)
                return 0
            lax.fori_loop(0, NBATCH, bstep, 0)
            plsc.subcore_barrier()

            def fl(j, _):
                r0 = my_row0 + j * FCH
                pltpu.sync_copy(acc.at[pl.ds(r0, FCH), :], fb)
                pltpu.sync_copy(fb, agg_out.at[c, pl.ds(r0, FCH), :])
                return 0
            lax.fori_loop(0, ROWS_PT // FCH, fl, 0)
            plsc.subcore_barrier()

        phase(xA, srcA, dstA, aggA)
        phase(xB, srcB, dstB, aggB)

    return pl.kernel(body, out_type=out_type, mesh=_mesh(),
                     scratch_types=scratch_types,
                     compiler_params=_SC_PARAMS)


def _make_sc_cnt():
    out_type = (jax.ShapeDtypeStruct((NPAD, 16), jnp.float32),) * 2

    scratch_types = (
        pltpu.VMEM((B,), jnp.int32),
        pltpu.VMEM((B, 16), jnp.float32),
        pltpu.VMEM((FCH, 16), jnp.float32),
        pltpu.VMEM((FCH, 16), jnp.float32),
        pltpu.VMEM_SHARED((NPAD, 16), jnp.float32),
    )

    def body(dstA, dstB, cntA, cntB, idx_d, ones_b, zb, fb, acc):
        c = lax.axis_index("c")
        s = lax.axis_index("s")
        my_row0 = s * ROWS_PT

        zeros16 = jnp.zeros((16,), jnp.float32)
        ones16 = jnp.ones((16,), jnp.float32)

        def initz(r, _):
            zb[r, :] = zeros16
            return 0
        lax.fori_loop(0, FCH, initz, 0)

        def initones(r, _):
            ones_b[r, :] = ones16
            return 0
        lax.fori_loop(0, B, initones, 0)

        def run(dst_hbm, cnt_out):
            def zchunk(j, _):
                pltpu.sync_copy(zb, acc.at[pl.ds(my_row0 + j * FCH, FCH), :])
                return 0
            lax.fori_loop(0, ROWS_PT // FCH, zchunk, 0)
            plsc.subcore_barrier()

            base = s * EPT

            def bstep(i, _):
                e0 = base + i * B
                pltpu.sync_copy(dst_hbm.at[pl.ds(e0, B)], idx_d)
                pltpu.sync_copy(ones_b, acc.at[idx_d], add=True)
                return 0
            lax.fori_loop(0, NBATCH, bstep, 0)
            plsc.subcore_barrier()

            def fl(j, _):
                r0 = my_row0 + j * FCH
                pltpu.sync_copy(acc.at[pl.ds(r0, FCH), :], fb)
                pltpu.sync_copy(fb, cnt_out.at[pl.ds(r0, FCH), :])
                return 0
            lax.fori_loop(0, ROWS_PT // FCH, fl, 0)

        @pl.when(c == 0)
        def _():
            run(dstA, cntA)

        @pl.when(c == 1)
        def _():
            run(dstB, cntB)

    return pl.kernel(body, out_type=out_type, mesh=_mesh(),
                     scratch_types=scratch_types,
                     compiler_params=_SC_PARAMS)


_sc_cache = {}


def _sc(name):
    fn = _sc_cache.get(name)
    if fn is None:
        fn = _make_sc_agg() if name == "agg" else _make_sc_cnt()
        _sc_cache[name] = fn
    return fn


_RB = 1000
_GRID = N // _RB

_CDIMS = (((1,), (1,)), ((), ()))


def _proj_body(me_ref, g_ref, w1_ref, w2_ref, b_ref, o_ref):
    out = lax.dot_general(me_ref[...], w1_ref[...], _CDIMS,
                          preferred_element_type=jnp.float32)
    out += lax.dot_general(g_ref[...], w2_ref[...], _CDIMS,
                           preferred_element_type=jnp.float32)
    o_ref[...] = out + b_ref[...]


def _proj(movie_emb, genres, w1, w2, b):
    return pl.pallas_call(
        _proj_body,
        grid=(_GRID,),
        in_specs=[
            pl.BlockSpec((_RB, D), lambda i: (i, 0)),
            pl.BlockSpec((_RB, NG), lambda i: (i, 0)),
            pl.BlockSpec((D, D), lambda i: (0, 0)),
            pl.BlockSpec((D, NG), lambda i: (0, 0)),
            pl.BlockSpec((1, D), lambda i: (0, 0)),
        ],
        out_specs=pl.BlockSpec((_RB, D), lambda i: (i, 0)),
        out_shape=jax.ShapeDtypeStruct((N, D), jnp.float32),
    )(movie_emb, genres, w1, w2, b)


def _combine_body(relu, alo_ref, ahi_ref, cnt_ref, xd_ref, wll_ref, wlh_ref,
                  wr_ref, b_ref, o_ref):
    inv = 1.0 / jnp.maximum(cnt_ref[...][:, 0:1], 1.0)
    out = lax.dot_general(alo_ref[0] * inv, wll_ref[...], _CDIMS,
                          preferred_element_type=jnp.float32)
    out += lax.dot_general(ahi_ref[0] * inv, wlh_ref[...], _CDIMS,
                           preferred_element_type=jnp.float32)
    out += lax.dot_general(xd_ref[...], wr_ref[...], _CDIMS,
                           preferred_element_type=jnp.float32)
    out += b_ref[...]
    if relu:
        out = jnp.maximum(out, 0.0)
    o_ref[...] = out


def _combine(agg, cnt, x_dst, Wl, Wr, b, relu):
    return pl.pallas_call(
        functools.partial(_combine_body, relu),
        grid=(_GRID,),
        in_specs=[
            pl.BlockSpec((1, _RB, HALF), lambda i: (0, i, 0)),
            pl.BlockSpec((1, _RB, HALF), lambda i: (1, i, 0)),
            pl.BlockSpec((_RB, 16), lambda i: (i, 0)),
            pl.BlockSpec((_RB, D), lambda i: (i, 0)),
            pl.BlockSpec((D, HALF), lambda i: (0, 0)),
            pl.BlockSpec((D, HALF), lambda i: (0, 0)),
            pl.BlockSpec((D, D), lambda i: (0, 0)),
            pl.BlockSpec((1, D), lambda i: (0, 0)),
        ],
        out_specs=pl.BlockSpec((_RB, D), lambda i: (i, 0)),
        out_shape=jax.ShapeDtypeStruct((N, D), jnp.float32),
    )(agg, agg, cnt, x_dst, Wl[:, :HALF], Wl[:, HALF:], Wr,
      b.reshape(1, D))


def _pad_idx(idx, fill):
    pad = jnp.full((E_PAD - E,), fill, jnp.int32)
    return jnp.concatenate([idx, pad])


def kernel(user_node_id, movie_node_id, movie_genres, edge_index_um,
           edge_index_mu, user_emb, movie_emb, proj_W, proj_b,
           conv1_um_Wl, conv1_um_Wr, conv1_um_b,
           conv1_mu_Wl, conv1_mu_Wr, conv1_mu_b,
           conv2_um_Wl, conv2_um_Wr, conv2_um_b,
           conv2_mu_Wl, conv2_mu_Wr, conv2_mu_b):
    xu0 = user_emb
    xm0 = _proj(movie_emb, movie_genres, proj_W[:, :D], proj_W[:, D:],
                proj_b.reshape(1, D))

    src_um = _pad_idx(edge_index_um[0], 0)
    dst_um = _pad_idx(edge_index_um[1], N)
    src_mu = _pad_idx(edge_index_mu[0], 0)
    dst_mu = _pad_idx(edge_index_mu[1], N)

    cnt_um, cnt_mu = _sc("cnt")(dst_um, dst_mu)

    agg_um1, agg_mu1 = _sc("agg")(
        xu0.reshape(2 * N, HALF), src_um, dst_um,
        xm0.reshape(2 * N, HALF), src_mu, dst_mu)

    xm1 = _combine(agg_um1, cnt_um, xm0, conv1_um_Wl, conv1_um_Wr,
                   conv1_um_b, relu=True)
    xu1 = _combine(agg_mu1, cnt_mu, xu0, conv1_mu_Wl, conv1_mu_Wr,
                   conv1_mu_b, relu=True)

    agg_um2, agg_mu2 = _sc("agg")(
        xu1.reshape(2 * N, HALF), src_um, dst_um,
        xm1.reshape(2 * N, HALF), src_mu, dst_mu)

    xm2 = _combine(agg_um2, cnt_um, xm1, conv2_um_Wl, conv2_um_Wr,
                   conv2_um_b, relu=False)
    xu2 = _combine(agg_mu2, cnt_mu, xu1, conv2_mu_Wl, conv2_mu_Wr,
                   conv2_mu_b, relu=False)
    return (xu2, xm2)
